# baseline (device time: 31148 ns/iter reference)
import jax
import jax.numpy as jnp
from jax import lax
from jax.experimental import pallas as pl
from jax.experimental.pallas import tpu as pltpu

N_DEV = 4
N_EXPERTS = 16
N_LOCAL_E = 4
CAP = 48
FLAT = 128


def _tdot(a, b):
    return lax.dot_general(a, b, (((0,), (0,)), ((), ())),
                           preferred_element_type=jnp.float32)


def kernel(x, router_W, route_idx, expert_W, shared_W):
    n, d = x.shape
    _, _, h = expert_W.shape
    chunk = n // N_DEV
    grp = N_LOCAL_E * CAP

    def body(x_ref, router_ref, idx_ref, expW_hbm, sharedW_hbm,
             out_ref, coeff_ref, w_vmem, sharedw_vmem, y_send, y_recv,
             send_sems, recv_sems, wload_sems, sload_sem):
        my_pos = lax.axis_index("i")

        w_copies = []
        for el in range(N_LOCAL_E):
            cp = pltpu.make_async_copy(
                expW_hbm.at[el], w_vmem.at[el], wload_sems.at[el])
            cp.start()
            w_copies.append(cp)
        s_copy = pltpu.make_async_copy(sharedW_hbm, sharedw_vmem, sload_sem)
        s_copy.start()

        barrier_sem = pltpu.get_barrier_semaphore()
        for dd in range(1, N_DEV):
            pl.semaphore_signal(
                barrier_sem, inc=1,
                device_id=(lax.rem(my_pos + dd, N_DEV),),
                device_id_type=pl.DeviceIdType.MESH,
            )

        xv = x_ref[:, :]
        scores = jnp.dot(xv, router_ref[:, :],
                         preferred_element_type=jnp.float32,
                         precision=lax.Precision.HIGHEST)
        scores = scores - jnp.max(scores, axis=-1, keepdims=True)
        ex = jnp.exp(scores)
        probs = ex / jnp.sum(ex, axis=-1, keepdims=True)
        lanes = lax.broadcasted_iota(jnp.int32, (n, N_EXPERTS), 1)
        gate = probs * (lanes == idx_ref[:, :]).astype(jnp.float32)
        r16 = lax.broadcasted_iota(jnp.int32, (N_EXPERTS, N_LOCAL_E), 0)
        c4 = lax.broadcasted_iota(jnp.int32, (N_EXPERTS, N_LOCAL_E), 1)
        sel = (r16 == N_LOCAL_E * my_pos + c4).astype(jnp.float32)
        coeff_ref[:, :] = jnp.dot(gate, sel, preferred_element_type=jnp.float32)

        tri_r = lax.broadcasted_iota(jnp.int32, (chunk, chunk), 0)
        tri_c = lax.broadcasted_iota(jnp.int32, (chunk, chunk), 1)
        t_strict = (tri_c < tri_r).astype(jnp.float32)
        cap_lane = lax.broadcasted_iota(jnp.int32, (chunk, CAP), 1)
        flat_lane = lax.broadcasted_iota(jnp.int32, (chunk, FLAT), 1)
        el_lane = lax.broadcasted_iota(jnp.int32, (chunk, N_LOCAL_E), 1)

        def build_E(row0, owner):
            idx_c = idx_ref[pl.ds(row0, chunk), :]
            masksf = (idx_c == N_LOCAL_E * owner + el_lane).astype(
                jnp.float32)
            flatmask = jnp.sum(masksf, axis=1, keepdims=True)
            ranks = jnp.dot(t_strict,
                            jnp.concatenate([masksf, flatmask], axis=1),
                            preferred_element_type=jnp.float32)
            ranks_i = ranks.astype(jnp.int32)
            blocks = []
            for el in range(N_LOCAL_E):
                hit = (cap_lane == ranks_i[:, el:el + 1]).astype(jnp.float32)
                blocks.append(hit * masksf[:, el:el + 1])
            e_blocked = jnp.concatenate(blocks, axis=1)
            e_flat = (flat_lane == ranks_i[:, N_LOCAL_E:]).astype(
                jnp.float32) * flatmask
            return e_blocked, e_flat

        e_blk, e_flt, xgs, cfgs = [], [], [], []
        for o in range(N_DEV):
            c = lax.rem(my_pos + o, N_DEV)
            row0 = c * chunk
            eb, ef = build_E(row0, my_pos)
            e_blk.append(eb)
            e_flt.append(ef)
            xgs.append(_tdot(eb, x_ref[pl.ds(row0, chunk), :]))
            cfgs.append(_tdot(eb, coeff_ref[pl.ds(row0, chunk), :]))

        y_blocks = {}
        for el in range(N_LOCAL_E):
            xga = jnp.concatenate(
                [xgs[o][el * CAP:(el + 1) * CAP, :] for o in range(N_DEV)],
                axis=0)
            cf_col = jnp.concatenate(
                [cfgs[o][el * CAP:(el + 1) * CAP, el:el + 1]
                 for o in range(N_DEV)], axis=0)
            w_copies[el].wait()
            y_el = jnp.dot(xga, w_vmem[el],
                           preferred_element_type=jnp.float32) * cf_col
            for o in range(N_DEV):
                y_blocks[(el, o)] = y_el[o * CAP:(o + 1) * CAP, :]

        rdmas = []
        for o in range(1, N_DEV):
            y_blocked = jnp.concatenate(
                [y_blocks[(el, o)] for el in range(N_LOCAL_E)], axis=0)
            r_perm = _tdot(e_flt[o], e_blk[o])
            y_send[o - 1, :, :] = jnp.dot(
                r_perm, y_blocked,
                preferred_element_type=jnp.float32).astype(jnp.bfloat16)
            if o == 1:
                pl.semaphore_wait(barrier_sem, N_DEV - 1)
            tgt = lax.rem(my_pos + o, N_DEV)
            rdma = pltpu.make_async_remote_copy(
                src_ref=y_send.at[o - 1],
                dst_ref=y_recv.at[o - 1],
                send_sem=send_sems.at[o - 1],
                recv_sem=recv_sems.at[o - 1],
                device_id=(tgt,), device_id_type=pl.DeviceIdType.MESH,
            )
            rdma.start()
            rdmas.append(rdma)

        own_y = jnp.concatenate(
            [y_blocks[(el, 0)] for el in range(N_LOCAL_E)], axis=0)
        acc = jnp.dot(e_blk[0], own_y, preferred_element_type=jnp.float32)
        s_copy.wait()
        acc = acc + jnp.dot(
            x_ref[pl.ds(my_pos * chunk, chunk), :], sharedw_vmem[:, :],
            preferred_element_type=jnp.float32,
        )

        for rdma in rdmas:
            rdma.wait()
        for j in range(N_DEV - 1):
            src = lax.rem(my_pos + N_DEV - (j + 1), N_DEV)
            _, e_src = build_E(my_pos * chunk, src)
            acc = acc + jnp.dot(
                e_src, y_recv[j, :, :].astype(jnp.float32),
                preferred_element_type=jnp.float32)
        out_ref[:, :] = acc

    return pl.pallas_call(
        body,
        out_shape=jax.ShapeDtypeStruct((chunk, h), jnp.float32),
        in_specs=[
            pl.BlockSpec(memory_space=pltpu.VMEM),
            pl.BlockSpec(memory_space=pltpu.VMEM),
            pl.BlockSpec(memory_space=pltpu.VMEM),
            pl.BlockSpec(memory_space=pltpu.MemorySpace.HBM),
            pl.BlockSpec(memory_space=pltpu.MemorySpace.HBM),
        ],
        out_specs=pl.BlockSpec(memory_space=pltpu.VMEM),
        scratch_shapes=[
            pltpu.VMEM((n, N_LOCAL_E), jnp.float32),
            pltpu.VMEM((N_LOCAL_E, d, h), jnp.float32),
            pltpu.VMEM((d, h), jnp.float32),
            pltpu.VMEM((3, FLAT, h), jnp.bfloat16),
            pltpu.VMEM((3, FLAT, h), jnp.bfloat16),
            pltpu.SemaphoreType.DMA((3,)),
            pltpu.SemaphoreType.DMA((3,)),
            pltpu.SemaphoreType.DMA((N_LOCAL_E,)),
            pltpu.SemaphoreType.DMA,
        ],
        compiler_params=pltpu.CompilerParams(collective_id=0),
    )(x, router_W, route_idx, expert_W, shared_W)


# device time: 25473 ns/iter; 1.2228x vs baseline; 1.2228x over previous
import jax
import jax.numpy as jnp
from jax import lax
from jax.experimental import pallas as pl
from jax.experimental.pallas import tpu as pltpu

N_DEV = 4
N_EXPERTS = 16
N_LOCAL_E = 4
K_CAP = 128


def _tdot(a, b):
    return lax.dot_general(a, b, (((0,), (0,)), ((), ())),
                           preferred_element_type=jnp.float32)


def kernel(x, router_W, route_idx, expert_W, shared_W):
    n, d = x.shape
    _, _, h = expert_W.shape
    chunk = n // N_DEV

    def body(x_ref, router_ref, idx_ref, expW_ref, sharedW_ref,
             out_ref, coeff_ref, y_send, y_recv,
             send_y_sems, recv_y_sems):
        my_pos = lax.axis_index("i")

        barrier_sem = pltpu.get_barrier_semaphore()
        for dd in range(1, N_DEV):
            pl.semaphore_signal(
                barrier_sem, inc=1,
                device_id=(lax.rem(my_pos + dd, N_DEV),),
                device_id_type=pl.DeviceIdType.MESH,
            )

        xv = x_ref[:, :]
        scores = jnp.dot(xv, router_ref[:, :],
                         preferred_element_type=jnp.float32,
                         precision=lax.Precision.HIGHEST)
        scores = scores - jnp.max(scores, axis=-1, keepdims=True)
        ex = jnp.exp(scores)
        probs = ex / jnp.sum(ex, axis=-1, keepdims=True)
        lanes = lax.broadcasted_iota(jnp.int32, (n, N_EXPERTS), 1)
        gate = probs * (lanes == idx_ref[:, :]).astype(jnp.float32)
        r16 = lax.broadcasted_iota(jnp.int32, (N_EXPERTS, N_LOCAL_E), 0)
        c4 = lax.broadcasted_iota(jnp.int32, (N_EXPERTS, N_LOCAL_E), 1)
        sel = (r16 == N_LOCAL_E * my_pos + c4).astype(jnp.float32)
        coeff_ref[:, :] = jnp.dot(gate, sel, preferred_element_type=jnp.float32)

        tri_r = lax.broadcasted_iota(jnp.int32, (chunk, chunk), 0)
        tri_c = lax.broadcasted_iota(jnp.int32, (chunk, chunk), 1)
        t_strict = (tri_c < tri_r).astype(jnp.float32)
        klane = lax.broadcasted_iota(jnp.int32, (chunk, K_CAP), 1)

        def compact(c, owner):
            idx_c = idx_ref[pl.ds(c * chunk, chunk), :]
            e_lo = N_LOCAL_E * owner
            maskf = ((idx_c >= e_lo) & (idx_c < e_lo + N_LOCAL_E)).astype(
                jnp.float32)
            rank = jnp.dot(t_strict, maskf,
                           preferred_element_type=jnp.float32)
            rank_i = rank.astype(jnp.int32)
            e_mat = (klane == rank_i).astype(jnp.float32) * maskf
            return e_mat

        def sparse_y(c, e_mat):
            xg = _tdot(e_mat, x_ref[pl.ds(c * chunk, chunk), :])
            cfg = _tdot(e_mat, coeff_ref[pl.ds(c * chunk, chunk), :])
            acc = jnp.zeros((K_CAP, h), jnp.float32)
            for el in range(N_LOCAL_E):
                y = jnp.dot(xg, expW_ref[el], preferred_element_type=jnp.float32)
                acc = acc + cfg[:, el:el + 1] * y
            return acc

        rdmas = {}
        for dd in (2, 1, 3):
            slot = dd - 1
            tgt = lax.rem(my_pos + dd, N_DEV)
            c = tgt
            e_mat = compact(c, my_pos)
            y_send[slot, :, :] = sparse_y(c, e_mat).astype(jnp.bfloat16)
            if dd == 2:
                pl.semaphore_wait(barrier_sem, N_DEV - 1)
            rdma_y = pltpu.make_async_remote_copy(
                src_ref=y_send.at[slot], dst_ref=y_recv.at[slot],
                send_sem=send_y_sems.at[slot], recv_sem=recv_y_sems.at[slot],
                device_id=(tgt,), device_id_type=pl.DeviceIdType.MESH,
            )
            rdma_y.start()
            rdmas[dd] = rdma_y

        e_own = compact(my_pos, my_pos)
        y_own = sparse_y(my_pos, e_own)
        own = jnp.dot(e_own, y_own, preferred_element_type=jnp.float32)
        shared_chunk = jnp.dot(
            x_ref[pl.ds(my_pos * chunk, chunk), :], sharedW_ref[:, :],
            preferred_element_type=jnp.float32,
        )
        acc = own + shared_chunk

        for dd in (2, 1, 3):
            rdmas[dd].wait()
            slot = dd - 1
            src = lax.rem(my_pos + N_DEV - dd, N_DEV)
            e_src = compact(my_pos, src)
            acc = acc + jnp.dot(
                e_src, y_recv[slot, :, :],
                preferred_element_type=jnp.float32,
            )
        out_ref[:, :] = acc

    return pl.pallas_call(
        body,
        out_shape=jax.ShapeDtypeStruct((chunk, h), jnp.float32),
        in_specs=[pl.BlockSpec(memory_space=pltpu.VMEM)] * 5,
        out_specs=pl.BlockSpec(memory_space=pltpu.VMEM),
        scratch_shapes=[
            pltpu.VMEM((n, N_LOCAL_E), jnp.float32),
            pltpu.VMEM((3, K_CAP, h), jnp.bfloat16),
            pltpu.VMEM((3, K_CAP, h), jnp.bfloat16),
            pltpu.SemaphoreType.DMA((3,)),
            pltpu.SemaphoreType.DMA((3,)),
        ],
        compiler_params=pltpu.CompilerParams(collective_id=0),
    )(x, router_W, route_idx, expert_W, shared_W)


# device time: 25432 ns/iter; 1.2248x vs baseline; 1.0016x over previous
import jax
import jax.numpy as jnp
from jax import lax
from jax.experimental import pallas as pl
from jax.experimental.pallas import tpu as pltpu

N_DEV = 4
N_EXPERTS = 16
N_LOCAL_E = 4
K_CAP = 128


def _tdot(a, b):
    return lax.dot_general(a, b, (((0,), (0,)), ((), ())),
                           preferred_element_type=jnp.float32)


def kernel(x, router_W, route_idx, expert_W, shared_W):
    n, d = x.shape
    _, _, h = expert_W.shape
    chunk = n // N_DEV

    def body(x_ref, router_ref, idx_ref, expW_ref, sharedW_hbm,
             out_ref, coeff_ref, sharedw_vmem, y_send, y_recv,
             send_y_sems, recv_y_sems, sload_sem):
        my_pos = lax.axis_index("i")

        s_copy = pltpu.make_async_copy(sharedW_hbm, sharedw_vmem, sload_sem)
        s_copy.start()

        barrier_sem = pltpu.get_barrier_semaphore()
        for dd in range(1, N_DEV):
            pl.semaphore_signal(
                barrier_sem, inc=1,
                device_id=(lax.rem(my_pos + dd, N_DEV),),
                device_id_type=pl.DeviceIdType.MESH,
            )

        xv = x_ref[:, :]
        scores = jnp.dot(xv, router_ref[:, :],
                         preferred_element_type=jnp.float32,
                         precision=lax.Precision.HIGHEST)
        scores = scores - jnp.max(scores, axis=-1, keepdims=True)
        ex = jnp.exp(scores)
        probs = ex / jnp.sum(ex, axis=-1, keepdims=True)
        lanes = lax.broadcasted_iota(jnp.int32, (n, N_EXPERTS), 1)
        gate = probs * (lanes == idx_ref[:, :]).astype(jnp.float32)
        r16 = lax.broadcasted_iota(jnp.int32, (N_EXPERTS, N_LOCAL_E), 0)
        c4 = lax.broadcasted_iota(jnp.int32, (N_EXPERTS, N_LOCAL_E), 1)
        sel = (r16 == N_LOCAL_E * my_pos + c4).astype(jnp.float32)
        coeff_ref[:, :] = jnp.dot(gate, sel, preferred_element_type=jnp.float32)

        tri_r = lax.broadcasted_iota(jnp.int32, (chunk, chunk), 0)
        tri_c = lax.broadcasted_iota(jnp.int32, (chunk, chunk), 1)
        t_strict = (tri_c < tri_r).astype(jnp.float32)
        klane = lax.broadcasted_iota(jnp.int32, (chunk, K_CAP), 1)

        def compact(c, owner):
            idx_c = idx_ref[pl.ds(c * chunk, chunk), :]
            e_lo = N_LOCAL_E * owner
            maskf = ((idx_c >= e_lo) & (idx_c < e_lo + N_LOCAL_E)).astype(
                jnp.float32)
            rank = jnp.dot(t_strict, maskf,
                           preferred_element_type=jnp.float32)
            rank_i = rank.astype(jnp.int32)
            e_mat = (klane == rank_i).astype(jnp.float32) * maskf
            return e_mat

        def sparse_y(c, e_mat):
            xg = _tdot(e_mat, x_ref[pl.ds(c * chunk, chunk), :])
            cfg = _tdot(e_mat, coeff_ref[pl.ds(c * chunk, chunk), :])
            acc = jnp.zeros((K_CAP, h), jnp.float32)
            for el in range(N_LOCAL_E):
                y = jnp.dot(xg, expW_ref[el], preferred_element_type=jnp.float32)
                acc = acc + cfg[:, el:el + 1] * y
            return acc

        rdmas = {}
        for dd in (2, 1, 3):
            slot = dd - 1
            tgt = lax.rem(my_pos + dd, N_DEV)
            c = tgt
            e_mat = compact(c, my_pos)
            y_send[slot, :, :] = sparse_y(c, e_mat).astype(jnp.bfloat16)
            if dd == 2:
                pl.semaphore_wait(barrier_sem, N_DEV - 1)
            rdma_y = pltpu.make_async_remote_copy(
                src_ref=y_send.at[slot], dst_ref=y_recv.at[slot],
                send_sem=send_y_sems.at[slot], recv_sem=recv_y_sems.at[slot],
                device_id=(tgt,), device_id_type=pl.DeviceIdType.MESH,
            )
            rdma_y.start()
            rdmas[dd] = rdma_y

        e_own = compact(my_pos, my_pos)
        y_own = sparse_y(my_pos, e_own)
        own = jnp.dot(e_own, y_own, preferred_element_type=jnp.float32)
        s_copy.wait()
        shared_chunk = jnp.dot(
            x_ref[pl.ds(my_pos * chunk, chunk), :], sharedw_vmem[:, :],
            preferred_element_type=jnp.float32,
        )
        acc = own + shared_chunk

        e_srcs = {}
        for dd in (2, 1, 3):
            src = lax.rem(my_pos + N_DEV - dd, N_DEV)
            e_srcs[dd] = compact(my_pos, src)

        for dd in (2, 1, 3):
            rdmas[dd].wait()
            slot = dd - 1
            acc = acc + jnp.dot(
                e_srcs[dd], y_recv[slot, :, :],
                preferred_element_type=jnp.float32,
            )
        out_ref[:, :] = acc

    return pl.pallas_call(
        body,
        out_shape=jax.ShapeDtypeStruct((chunk, h), jnp.float32),
        in_specs=[
            pl.BlockSpec(memory_space=pltpu.VMEM),
            pl.BlockSpec(memory_space=pltpu.VMEM),
            pl.BlockSpec(memory_space=pltpu.VMEM),
            pl.BlockSpec(memory_space=pltpu.VMEM),
            pl.BlockSpec(memory_space=pltpu.MemorySpace.HBM),
        ],
        out_specs=pl.BlockSpec(memory_space=pltpu.VMEM),
        scratch_shapes=[
            pltpu.VMEM((n, N_LOCAL_E), jnp.float32),
            pltpu.VMEM((d, h), jnp.float32),
            pltpu.VMEM((3, K_CAP, h), jnp.bfloat16),
            pltpu.VMEM((3, K_CAP, h), jnp.bfloat16),
            pltpu.SemaphoreType.DMA((3,)),
            pltpu.SemaphoreType.DMA((3,)),
            pltpu.SemaphoreType.DMA,
        ],
        compiler_params=pltpu.CompilerParams(collective_id=0),
    )(x, router_W, route_idx, expert_W, shared_W)
